# baseline (speedup 1.0000x reference)
import functools

import jax
import jax.numpy as jnp
from jax import lax
from jax.experimental import pallas as pl
from jax.experimental.pallas import tpu as pltpu
from jax.experimental.pallas import tpu_sc as plsc

_EMBED = 32
_CHUNK = 512
_RANGE = 65536
_ROWPAD = 128
_HITBUF = 128


def _popcount(mask):
    return plsc.all_reduce_population_count(mask)[0]


def _build_stage1(batch, n_rows):
    info = plsc.get_sparse_core_info()
    nc, ns, lanes = info.num_cores, info.num_subcores, info.num_lanes
    rows_per_w = _RANGE
    n_full = (n_rows // _CHUNK) * _CHUNK
    tail = n_rows - n_full
    mesh = plsc.VectorSubcoreMesh(core_axis_name="c", subcore_axis_name="s")

    @functools.partial(
        pl.kernel,
        mesh=mesh,
        out_type=(
            jax.ShapeDtypeStruct((batch, _ROWPAD), jnp.float32),
            jax.ShapeDtypeStruct((batch, _ROWPAD), jnp.float32),
        ),
        scratch_types=[
            pltpu.VMEM((2048,), jnp.int32),
            pltpu.VMEM((batch + 16,), jnp.int32),
            pltpu.VMEM((batch + 16,), jnp.int32),
            pltpu.VMEM((_EMBED, _CHUNK), jnp.float32),
            pltpu.VMEM((tail, _EMBED), jnp.float32),
            pltpu.VMEM((_HITBUF, _ROWPAD), jnp.float32),
            pltpu.VMEM((_HITBUF + 16,), jnp.int32),
            pltpu.VMEM((_HITBUF,), jnp.int32),
            pltpu.SemaphoreType.DMA,
        ],
        compiler_params=pltpu.CompilerParams(needs_layout_passes=False),
    )
    def stage1(idx2_hbm, ut_hbm, mt_hbm, utail_hbm, mtail_hbm,
               urowsp_hbm, mrowsp_hbm,
               idx_v, h1i_v, h1s_v, chunk_v, tail_v, hrow_v, hslot_v,
               hslot2_v, sem):
        core = lax.axis_index("c")
        sid = lax.axis_index("s")
        lo = sid * rows_per_w
        hi = jnp.minimum(lo + rows_per_w, n_rows)
        iota = lax.iota(jnp.int32, lanes)

        def run(tab_hbm, tail_hbm, out_hbm):
            pltpu.sync_copy(tail_hbm, tail_v)

            def l1_blk(b, n1_in):
                pltpu.sync_copy(idx2_hbm.at[core, pl.ds(b * 2048, 2048)],
                                idx_v)

                def l1_body(k, n1):
                    v = idx_v[pl.ds(k * lanes, lanes)]
                    slots = b * 2048 + k * lanes + iota
                    mask = (v >= lo) & (v < hi)
                    cnt = _popcount(mask)
                    plsc.store_compressed(
                        h1i_v.at[pl.ds(n1, lanes)], v, mask=mask)
                    plsc.store_compressed(
                        h1s_v.at[pl.ds(n1, lanes)], slots, mask=mask)
                    return n1 + cnt

                return lax.fori_loop(0, 2048 // lanes, l1_body, n1_in)

            n1 = lax.fori_loop(0, batch // 2048, l1_blk, 0)

            def flush(nh):
                @pl.when(nh > 0)
                def _():
                    def pad_body(q, _):
                        cur = hslot_v[pl.ds(q * lanes, lanes)]
                        hslot2_v[pl.ds(q * lanes, lanes)] = jnp.where(
                            q * lanes + iota >= nh, -1, cur)
                        return _

                    lax.fori_loop(0, _HITBUF // lanes, pad_body, 0)
                    pltpu.async_copy(
                        hrow_v,
                        out_hbm.at[plsc.Indices(hslot2_v, ignored_value=-1)],
                        sem).wait()

            n_chunks = rows_per_w // _CHUNK

            def chunk_body(cc, nh):
                base = lo + cc * _CHUNK

                @pl.when(base < n_full)
                def _():
                    pltpu.sync_copy(
                        tab_hbm.at[:, pl.ds(base, _CHUNK)], chunk_v)

                def l2_body(j, carry):
                    nh_in = carry
                    v = h1i_v[pl.ds(j * lanes, lanes)]
                    sl = h1s_v[pl.ds(j * lanes, lanes)]
                    valid = (j * lanes + iota) < n1
                    inch = valid & (v >= base) & (v < base + _CHUNK) & (
                        base < n_full)
                    intail = valid & (v >= n_full) & (cc == 0)

                    inch_i = inch.astype(jnp.int32)
                    intail_i = intail.astype(jnp.int32)

                    def extract(k, nh_k):
                        m_ch = inch_i[k] == 1
                        m_tl = intail_i[k] == 1

                        @pl.when(m_ch | m_tl)
                        def _():
                            r = v[k]
                            rc = jnp.where(m_ch, r - base, 0)
                            g0 = plsc.load_gather(
                                chunk_v, [iota, jnp.full((lanes,), rc,
                                                         jnp.int32)])
                            g1 = plsc.load_gather(
                                chunk_v, [iota + lanes,
                                          jnp.full((lanes,), rc, jnp.int32)])
                            rt = jnp.where(m_tl, r - n_full, 0)
                            t0 = plsc.load_gather(
                                tail_v, [jnp.full((lanes,), rt, jnp.int32),
                                         iota])
                            t1 = plsc.load_gather(
                                tail_v, [jnp.full((lanes,), rt, jnp.int32),
                                         iota + lanes])
                            r0 = jnp.where(m_tl, t0, g0)
                            r1 = jnp.where(m_tl, t1, g1)
                            hrow_v[nh_k, pl.ds(0, lanes)] = r0
                            hrow_v[nh_k, pl.ds(lanes, lanes)] = r1
                            plsc.store_compressed(
                                hslot_v.at[pl.ds(nh_k, lanes)], sl,
                                mask=iota == k)

                        return nh_k + jnp.where(m_ch | m_tl, 1, 0)

                    nh_out = nh_in
                    for k in range(lanes):
                        nh_out = extract(k, nh_out)
                    return nh_out

                def maybe_flush(nh_now):
                    @pl.when(nh_now > _HITBUF - lanes)
                    def _():
                        flush(nh_now)
                    return jnp.where(nh_now > _HITBUF - lanes, 0, nh_now)

                def l2_outer(j, nh_in):
                    nh_mid = l2_body(j, nh_in)
                    return maybe_flush(nh_mid)

                n_j = (n1 + lanes - 1) // lanes
                return lax.fori_loop(0, n_j, l2_outer, nh)

            nh_final = lax.fori_loop(0, n_chunks, chunk_body, 0)
            flush(nh_final)

        @pl.when(core == 0)
        def _():
            run(ut_hbm, utail_hbm, urowsp_hbm)

        @pl.when(core == 1)
        def _():
            run(mt_hbm, mtail_hbm, mrowsp_hbm)

    return stage1


def _build_stage2(batch):
    info = plsc.get_sparse_core_info()
    nc, ns, lanes = info.num_cores, info.num_subcores, info.num_lanes
    nw = nc * ns
    b_per_w = batch // nw
    blk = 128
    mesh = plsc.VectorSubcoreMesh(core_axis_name="c", subcore_axis_name="s")

    @functools.partial(
        pl.kernel,
        mesh=mesh,
        out_type=jax.ShapeDtypeStruct((batch,), jnp.float32),
        scratch_types=[
            pltpu.VMEM((blk, _ROWPAD), jnp.float32),
            pltpu.VMEM((blk, _ROWPAD), jnp.float32),
            pltpu.VMEM((b_per_w,), jnp.float32),
            pltpu.SemaphoreType.DMA,
        ],
        compiler_params=pltpu.CompilerParams(needs_layout_passes=False),
    )
    def stage2(urowsp_hbm, mrowsp_hbm, out_hbm, u_v, m_v, out_v, sem):
        wid = lax.axis_index("s") * nc + lax.axis_index("c")
        base = wid * b_per_w
        iota = lax.iota(jnp.int32, lanes)

        for sub in range(b_per_w // blk):
            pltpu.sync_copy(urowsp_hbm.at[pl.ds(base + sub * blk, blk)], u_v)
            pltpu.sync_copy(mrowsp_hbm.at[pl.ds(base + sub * blk, blk)], m_v)

            def grp_body(g, carry):
                rows = g * lanes + iota
                acc = jnp.zeros((lanes,), jnp.float32)
                for d in range(_EMBED):
                    col = jnp.full((lanes,), d, jnp.int32)
                    acc = acc + (plsc.load_gather(u_v, [rows, col])
                                 * plsc.load_gather(m_v, [rows, col]))
                out_v[pl.ds(sub * blk + g * lanes, lanes)] = acc
                return carry

            lax.fori_loop(0, blk // lanes, grp_body, 0)

        pltpu.sync_copy(out_v, out_hbm.at[pl.ds(base, b_per_w)])

    return stage2


def kernel(user_movie_pair, user_embeddings, movie_embeddings):
    batch = user_movie_pair.shape[0]
    n_rows = user_embeddings.shape[0]
    n_full = (n_rows // _CHUNK) * _CHUNK
    pair = user_movie_pair.astype(jnp.int32)
    idx2 = jnp.stack([pair[:, 0], pair[:, 1]])
    ut = user_embeddings.T
    mt = movie_embeddings.T
    utail = user_embeddings[n_full:]
    mtail = movie_embeddings[n_full:]
    stage1 = _build_stage1(batch, n_rows)
    urowsp, mrowsp = stage1(idx2, ut, mt, utail, mtail)
    stage2 = _build_stage2(batch)
    out = stage2(urowsp, mrowsp)
    return out.reshape(batch, 1)


# speedup vs baseline: 4.3486x; 4.3486x over previous
import functools

import jax
import jax.numpy as jnp
from jax import lax
from jax.experimental import pallas as pl
from jax.experimental.pallas import tpu as pltpu
from jax.experimental.pallas import tpu_sc as plsc

_EMBED = 32
_CHUNK = 512
_RANGE = 65536
_ROWPAD = 128
_HITBUF = 128


def _popcount(mask):
    return plsc.all_reduce_population_count(mask)[0]


def _build_stage1(batch, n_rows):
    info = plsc.get_sparse_core_info()
    nc, ns, lanes = info.num_cores, info.num_subcores, info.num_lanes
    rows_per_w = _RANGE
    n_full = (n_rows // _CHUNK) * _CHUNK
    tail = n_rows - n_full
    mesh = plsc.VectorSubcoreMesh(core_axis_name="c", subcore_axis_name="s")

    @functools.partial(
        pl.kernel,
        mesh=mesh,
        out_type=(
            jax.ShapeDtypeStruct((batch, _ROWPAD), jnp.float32),
            jax.ShapeDtypeStruct((batch, _ROWPAD), jnp.float32),
        ),
        scratch_types=[
            pltpu.VMEM((2048,), jnp.int32),
            pltpu.VMEM((batch + 16,), jnp.int32),
            pltpu.VMEM((batch + 16,), jnp.int32),
            pltpu.VMEM((2, _EMBED, _CHUNK), jnp.float32),
            pltpu.VMEM((tail, _EMBED), jnp.float32),
            pltpu.VMEM((_HITBUF, _ROWPAD), jnp.float32),
            pltpu.VMEM((_HITBUF + 16,), jnp.int32),
            pltpu.VMEM((_HITBUF,), jnp.int32),
            pltpu.SemaphoreType.DMA,
        ],
        compiler_params=pltpu.CompilerParams(needs_layout_passes=False),
    )
    def stage1(idx2_hbm, ut_hbm, mt_hbm, utail_hbm, mtail_hbm,
               urowsp_hbm, mrowsp_hbm,
               idx_v, h1i_v, h1s_v, chunk_v, tail_v, hrow_v, hslot_v,
               hslot2_v, sem):
        core = lax.axis_index("c")
        sid = lax.axis_index("s")
        lo = sid * rows_per_w
        hi = jnp.minimum(lo + rows_per_w, n_rows)
        iota = lax.iota(jnp.int32, lanes)

        def run(tab_hbm, tail_hbm, out_hbm):
            pltpu.sync_copy(tail_hbm, tail_v)

            def l1_blk(b, n1_in):
                pltpu.sync_copy(idx2_hbm.at[core, pl.ds(b * 2048, 2048)],
                                idx_v)

                def l1_body(k, n1):
                    v = idx_v[pl.ds(k * lanes, lanes)]
                    slots = b * 2048 + k * lanes + iota
                    mask = (v >= lo) & (v < hi)
                    cnt = _popcount(mask)
                    plsc.store_compressed(
                        h1i_v.at[pl.ds(n1, lanes)], v, mask=mask)
                    plsc.store_compressed(
                        h1s_v.at[pl.ds(n1, lanes)], slots, mask=mask)
                    return n1 + cnt

                return lax.fori_loop(0, 2048 // lanes, l1_body, n1_in)

            n1 = lax.fori_loop(0, batch // 2048, l1_blk, 0)

            def flush(nh):
                @pl.when(nh > 0)
                def _():
                    def pad_body(q, _):
                        cur = hslot_v[pl.ds(q * lanes, lanes)]
                        hslot2_v[pl.ds(q * lanes, lanes)] = jnp.where(
                            q * lanes + iota >= nh, -1, cur)
                        return _

                    lax.fori_loop(0, _HITBUF // lanes, pad_body, 0)
                    pltpu.async_copy(
                        hrow_v,
                        out_hbm.at[plsc.Indices(hslot2_v, ignored_value=-1)],
                        sem).wait()

            n_chunks = rows_per_w // _CHUNK

            def fetch(cc, buf):
                base = lo + cc * _CHUNK

                @pl.when(base < n_full)
                def _():
                    pltpu.async_copy(
                        tab_hbm.at[:, pl.ds(base, _CHUNK)],
                        chunk_v.at[buf], sem)

            def wait_fetch(cc, buf):
                base = lo + cc * _CHUNK

                @pl.when(base < n_full)
                def _():
                    pltpu.make_async_copy(
                        tab_hbm.at[:, pl.ds(lo, _CHUNK)],
                        chunk_v.at[buf], sem).wait()

            fetch(0, 0)

            def chunk_body(cc, nh):
                base = lo + cc * _CHUNK
                buf = lax.rem(cc, 2)

                @pl.when(cc + 1 < n_chunks)
                def _():
                    fetch(cc + 1, 1 - buf)

                wait_fetch(cc, buf)
                cbuf = chunk_v.at[buf]

                def extract_group(v, sl, inch, intail, nh_in):
                    inch_i = inch.astype(jnp.int32)
                    intail_i = intail.astype(jnp.int32)

                    def extract(k, nh_k):
                        m_ch = inch_i[k] == 1
                        m_tl = intail_i[k] == 1

                        @pl.when(m_ch | m_tl)
                        def _():
                            r = v[k]
                            rc = jnp.where(m_ch, r - base, 0)
                            g0 = plsc.load_gather(
                                cbuf, [iota, jnp.full((lanes,), rc,
                                                      jnp.int32)])
                            g1 = plsc.load_gather(
                                cbuf, [iota + lanes,
                                       jnp.full((lanes,), rc, jnp.int32)])
                            rt = jnp.where(m_tl, r - n_full, 0)
                            t0 = plsc.load_gather(
                                tail_v, [jnp.full((lanes,), rt, jnp.int32),
                                         iota])
                            t1 = plsc.load_gather(
                                tail_v, [jnp.full((lanes,), rt, jnp.int32),
                                         iota + lanes])
                            r0 = jnp.where(m_tl, t0, g0)
                            r1 = jnp.where(m_tl, t1, g1)
                            hrow_v[nh_k, pl.ds(0, lanes)] = r0
                            hrow_v[nh_k, pl.ds(lanes, lanes)] = r1
                            plsc.store_compressed(
                                hslot_v.at[pl.ds(nh_k, lanes)], sl,
                                mask=iota == k)

                        return nh_k + jnp.where(m_ch | m_tl, 1, 0)

                    nh_out = nh_in
                    for k in range(lanes):
                        nh_out = extract(k, nh_out)

                    @pl.when(nh_out > _HITBUF - lanes)
                    def _():
                        flush(nh_out)

                    return jnp.where(nh_out > _HITBUF - lanes, 0, nh_out)

                def l2_body(j, nh_in):
                    v = h1i_v[pl.ds(j * lanes, lanes)]
                    sl = h1s_v[pl.ds(j * lanes, lanes)]
                    valid = (j * lanes + iota) < n1
                    inch = valid & (v >= base) & (v < base + _CHUNK) & (
                        base < n_full)
                    intail = valid & (v >= n_full) & (cc == 0)
                    any_hit = _popcount(inch | intail)
                    return lax.cond(
                        any_hit > 0,
                        lambda: extract_group(v, sl, inch, intail, nh_in),
                        lambda: nh_in)

                n_j = (n1 + lanes - 1) // lanes
                return lax.fori_loop(0, n_j, l2_body, nh)

            nh_final = lax.fori_loop(0, n_chunks, chunk_body, 0)
            flush(nh_final)

        @pl.when(core == 0)
        def _():
            run(ut_hbm, utail_hbm, urowsp_hbm)

        @pl.when(core == 1)
        def _():
            run(mt_hbm, mtail_hbm, mrowsp_hbm)

    return stage1


def _build_stage2(batch):
    info = plsc.get_sparse_core_info()
    nc, ns, lanes = info.num_cores, info.num_subcores, info.num_lanes
    nw = nc * ns
    b_per_w = batch // nw
    blk = 128
    mesh = plsc.VectorSubcoreMesh(core_axis_name="c", subcore_axis_name="s")

    @functools.partial(
        pl.kernel,
        mesh=mesh,
        out_type=jax.ShapeDtypeStruct((batch,), jnp.float32),
        scratch_types=[
            pltpu.VMEM((blk, _ROWPAD), jnp.float32),
            pltpu.VMEM((blk, _ROWPAD), jnp.float32),
            pltpu.VMEM((b_per_w,), jnp.float32),
            pltpu.SemaphoreType.DMA,
        ],
        compiler_params=pltpu.CompilerParams(needs_layout_passes=False),
    )
    def stage2(urowsp_hbm, mrowsp_hbm, out_hbm, u_v, m_v, out_v, sem):
        wid = lax.axis_index("s") * nc + lax.axis_index("c")
        base = wid * b_per_w
        iota = lax.iota(jnp.int32, lanes)

        for sub in range(b_per_w // blk):
            pltpu.sync_copy(urowsp_hbm.at[pl.ds(base + sub * blk, blk)], u_v)
            pltpu.sync_copy(mrowsp_hbm.at[pl.ds(base + sub * blk, blk)], m_v)

            def grp_body(g, carry):
                rows = g * lanes + iota
                acc = jnp.zeros((lanes,), jnp.float32)
                for d in range(_EMBED):
                    col = jnp.full((lanes,), d, jnp.int32)
                    acc = acc + (plsc.load_gather(u_v, [rows, col])
                                 * plsc.load_gather(m_v, [rows, col]))
                out_v[pl.ds(sub * blk + g * lanes, lanes)] = acc
                return carry

            lax.fori_loop(0, blk // lanes, grp_body, 0)

        pltpu.sync_copy(out_v, out_hbm.at[pl.ds(base, b_per_w)])

    return stage2


def kernel(user_movie_pair, user_embeddings, movie_embeddings):
    batch = user_movie_pair.shape[0]
    n_rows = user_embeddings.shape[0]
    n_full = (n_rows // _CHUNK) * _CHUNK
    pair = user_movie_pair.astype(jnp.int32)
    idx2 = jnp.stack([pair[:, 0], pair[:, 1]])
    ut = user_embeddings.T
    mt = movie_embeddings.T
    utail = user_embeddings[n_full:]
    mtail = movie_embeddings[n_full:]
    stage1 = _build_stage1(batch, n_rows)
    urowsp, mrowsp = stage1(idx2, ut, mt, utail, mtail)
    stage2 = _build_stage2(batch)
    out = stage2(urowsp, mrowsp)
    return out.reshape(batch, 1)


# speedup vs baseline: 8.5621x; 1.9689x over previous
import functools

import jax
import jax.numpy as jnp
from jax import lax
from jax.experimental import pallas as pl
from jax.experimental.pallas import tpu as pltpu
from jax.experimental.pallas import tpu_sc as plsc

_EMBED = 32
_CHUNK = 512
_RANGE = 65536
_ROWPAD = 128
_HITBUF = 128


def _popcount(mask):
    return plsc.all_reduce_population_count(mask)[0]


def _build_stage1(batch, n_rows):
    info = plsc.get_sparse_core_info()
    nc, ns, lanes = info.num_cores, info.num_subcores, info.num_lanes
    rows_per_w = _RANGE
    n_full = (n_rows // _CHUNK) * _CHUNK
    tail = n_rows - n_full
    mesh = plsc.VectorSubcoreMesh(core_axis_name="c", subcore_axis_name="s")

    @functools.partial(
        pl.kernel,
        mesh=mesh,
        out_type=(
            jax.ShapeDtypeStruct((batch, _ROWPAD), jnp.float32),
            jax.ShapeDtypeStruct((batch, _ROWPAD), jnp.float32),
        ),
        scratch_types=[
            pltpu.VMEM((2048,), jnp.int32),
            pltpu.VMEM((batch + 16,), jnp.int32),
            pltpu.VMEM((batch + 16,), jnp.int32),
            pltpu.VMEM((2, _EMBED, _CHUNK), jnp.float32),
            pltpu.VMEM((tail, _EMBED), jnp.float32),
            pltpu.VMEM((_HITBUF, _ROWPAD), jnp.float32),
            pltpu.VMEM((_HITBUF + 16,), jnp.int32),
            pltpu.VMEM((_HITBUF,), jnp.int32),
            pltpu.VMEM((batch + 16,), jnp.int32),
            pltpu.VMEM((batch + 16,), jnp.int32),
            pltpu.SemaphoreType.DMA,
        ],
        compiler_params=pltpu.CompilerParams(needs_layout_passes=False),
    )
    def stage1(idx2_hbm, ut_hbm, mt_hbm, utail_hbm, mtail_hbm,
               urowsp_hbm, mrowsp_hbm,
               idx_v, h1i_v, h1s_v, chunk_v, tail_v, hrow_v, hslot_v,
               hslot2_v, h2i_v, h2s_v, sem):
        core = lax.axis_index("c")
        sid = lax.axis_index("s")
        lo = sid * rows_per_w
        hi = jnp.minimum(lo + rows_per_w, n_rows)
        iota = lax.iota(jnp.int32, lanes)

        def run(tab_hbm, tail_hbm, out_hbm):
            pltpu.sync_copy(tail_hbm, tail_v)

            def l1_blk(b, n1_in):
                pltpu.sync_copy(idx2_hbm.at[core, pl.ds(b * 2048, 2048)],
                                idx_v)

                def l1_body(k, n1):
                    v = idx_v[pl.ds(k * lanes, lanes)]
                    slots = b * 2048 + k * lanes + iota
                    mask = (v >= lo) & (v < hi)
                    cnt = _popcount(mask)
                    plsc.store_compressed(
                        h1i_v.at[pl.ds(n1, lanes)], v, mask=mask)
                    plsc.store_compressed(
                        h1s_v.at[pl.ds(n1, lanes)], slots, mask=mask)
                    return n1 + cnt

                return lax.fori_loop(0, 2048 // lanes, l1_body, n1_in)

            n1 = lax.fori_loop(0, batch // 2048, l1_blk, 0)

            def flush(nh):
                @pl.when(nh > 0)
                def _():
                    def pad_body(q, _):
                        cur = hslot_v[pl.ds(q * lanes, lanes)]
                        hslot2_v[pl.ds(q * lanes, lanes)] = jnp.where(
                            q * lanes + iota >= nh, -1, cur)
                        return _

                    lax.fori_loop(0, _HITBUF // lanes, pad_body, 0)
                    pltpu.async_copy(
                        hrow_v,
                        out_hbm.at[plsc.Indices(hslot2_v, ignored_value=-1)],
                        sem).wait()

            n_chunks = rows_per_w // _CHUNK

            def fetch(cc, buf):
                base = lo + cc * _CHUNK

                @pl.when(base < n_full)
                def _():
                    pltpu.async_copy(
                        tab_hbm.at[:, pl.ds(base, _CHUNK)],
                        chunk_v.at[buf], sem)

            def wait_fetch(cc, buf):
                base = lo + cc * _CHUNK

                @pl.when(base < n_full)
                def _():
                    pltpu.make_async_copy(
                        tab_hbm.at[:, pl.ds(lo, _CHUNK)],
                        chunk_v.at[buf], sem).wait()

            def tail_scan(j, nh_in):
                v = h1i_v[pl.ds(j * lanes, lanes)]
                sl = h1s_v[pl.ds(j * lanes, lanes)]
                valid = (j * lanes + iota) < n1
                intail = valid & (v >= n_full)
                i_tl = intail.astype(jnp.int32)

                def extract(k, nh_k):
                    m_tl = i_tl[k] == 1

                    @pl.when(m_tl)
                    def _():
                        rt = v[k] - n_full
                        t0 = plsc.load_gather(
                            tail_v, [jnp.full((lanes,), rt, jnp.int32),
                                     iota])
                        t1 = plsc.load_gather(
                            tail_v, [jnp.full((lanes,), rt, jnp.int32),
                                     iota + lanes])
                        hrow_v[nh_k, pl.ds(0, lanes)] = t0
                        hrow_v[nh_k, pl.ds(lanes, lanes)] = t1
                        plsc.store_compressed(
                            hslot_v.at[pl.ds(nh_k, lanes)], sl,
                            mask=iota == k)

                    return nh_k + jnp.where(m_tl, 1, 0)

                def run_group():
                    nh_out = nh_in
                    for k in range(lanes):
                        nh_out = extract(k, nh_out)

                    @pl.when(nh_out > _HITBUF - lanes)
                    def _():
                        flush(nh_out)

                    return jnp.where(nh_out > _HITBUF - lanes, 0, nh_out)

                return lax.cond(_popcount(intail) > 0, run_group,
                                lambda: nh_in)

            n_j0 = (n1 + lanes - 1) // lanes
            nh_tail = lax.fori_loop(0, n_j0, tail_scan, 0)

            fetch(0, 0)

            def chunk_body(cc, nh):
                base = lo + cc * _CHUNK
                buf = lax.rem(cc, 2)

                @pl.when(cc + 1 < n_chunks)
                def _():
                    fetch(cc + 1, 1 - buf)

                wait_fetch(cc, buf)
                cbuf = chunk_v.at[buf]
                in_range = base < n_full

                def collect(j, n2):
                    v = h1i_v[pl.ds(j * lanes, lanes)]
                    sl = h1s_v[pl.ds(j * lanes, lanes)]
                    valid = (j * lanes + iota) < n1
                    inch = valid & (v >= base) & (v < base + _CHUNK) & in_range
                    cnt = _popcount(inch)

                    @pl.when(cnt > 0)
                    def _():
                        plsc.store_compressed(
                            h2i_v.at[pl.ds(n2, lanes)], v - base, mask=inch)
                        plsc.store_compressed(
                            h2s_v.at[pl.ds(n2, lanes)], sl, mask=inch)

                    return n2 + cnt

                n_j = (n1 + lanes - 1) // lanes
                n2 = lax.fori_loop(0, n_j, collect, 0)

                def ext_group(g, nh_in):
                    v = h2i_v[pl.ds(g * lanes, lanes)]
                    sl = h2s_v[pl.ds(g * lanes, lanes)]
                    valid_i = ((g * lanes + iota) < n2).astype(jnp.int32)

                    def extract(k, nh_k):
                        m_ch = valid_i[k] == 1

                        @pl.when(m_ch)
                        def _():
                            rc = v[k]
                            rc_vec = jnp.full((lanes,), rc, jnp.int32)
                            g0 = plsc.load_gather(cbuf, [iota, rc_vec])
                            g1 = plsc.load_gather(
                                cbuf, [iota + lanes, rc_vec])
                            hrow_v[nh_k, pl.ds(0, lanes)] = g0
                            hrow_v[nh_k, pl.ds(lanes, lanes)] = g1
                            plsc.store_compressed(
                                hslot_v.at[pl.ds(nh_k, lanes)], sl,
                                mask=iota == k)

                        return nh_k + jnp.where(m_ch, 1, 0)

                    nh_out = nh_in
                    for k in range(lanes):
                        nh_out = extract(k, nh_out)

                    @pl.when(nh_out > _HITBUF - lanes)
                    def _():
                        flush(nh_out)

                    return jnp.where(nh_out > _HITBUF - lanes, 0, nh_out)

                n_g = (n2 + lanes - 1) // lanes
                return lax.fori_loop(0, n_g, ext_group, nh)

            nh_final = lax.fori_loop(0, n_chunks, chunk_body, nh_tail)
            flush(nh_final)

        @pl.when(core == 0)
        def _():
            run(ut_hbm, utail_hbm, urowsp_hbm)

        @pl.when(core == 1)
        def _():
            run(mt_hbm, mtail_hbm, mrowsp_hbm)

    return stage1


def _build_stage2(batch):
    info = plsc.get_sparse_core_info()
    nc, ns, lanes = info.num_cores, info.num_subcores, info.num_lanes
    nw = nc * ns
    b_per_w = batch // nw
    blk = 128
    mesh = plsc.VectorSubcoreMesh(core_axis_name="c", subcore_axis_name="s")

    @functools.partial(
        pl.kernel,
        mesh=mesh,
        out_type=jax.ShapeDtypeStruct((batch,), jnp.float32),
        scratch_types=[
            pltpu.VMEM((blk, _ROWPAD), jnp.float32),
            pltpu.VMEM((blk, _ROWPAD), jnp.float32),
            pltpu.VMEM((b_per_w,), jnp.float32),
            pltpu.SemaphoreType.DMA,
        ],
        compiler_params=pltpu.CompilerParams(needs_layout_passes=False),
    )
    def stage2(urowsp_hbm, mrowsp_hbm, out_hbm, u_v, m_v, out_v, sem):
        wid = lax.axis_index("s") * nc + lax.axis_index("c")
        base = wid * b_per_w
        iota = lax.iota(jnp.int32, lanes)

        for sub in range(b_per_w // blk):
            pltpu.sync_copy(urowsp_hbm.at[pl.ds(base + sub * blk, blk)], u_v)
            pltpu.sync_copy(mrowsp_hbm.at[pl.ds(base + sub * blk, blk)], m_v)

            def grp_body(g, carry):
                rows = g * lanes + iota
                acc = jnp.zeros((lanes,), jnp.float32)
                for d in range(_EMBED):
                    col = jnp.full((lanes,), d, jnp.int32)
                    acc = acc + (plsc.load_gather(u_v, [rows, col])
                                 * plsc.load_gather(m_v, [rows, col]))
                out_v[pl.ds(sub * blk + g * lanes, lanes)] = acc
                return carry

            lax.fori_loop(0, blk // lanes, grp_body, 0)

        pltpu.sync_copy(out_v, out_hbm.at[pl.ds(base, b_per_w)])

    return stage2


def kernel(user_movie_pair, user_embeddings, movie_embeddings):
    batch = user_movie_pair.shape[0]
    n_rows = user_embeddings.shape[0]
    n_full = (n_rows // _CHUNK) * _CHUNK
    pair = user_movie_pair.astype(jnp.int32)
    idx2 = jnp.stack([pair[:, 0], pair[:, 1]])
    ut = user_embeddings.T
    mt = movie_embeddings.T
    utail = user_embeddings[n_full:]
    mtail = movie_embeddings[n_full:]
    stage1 = _build_stage1(batch, n_rows)
    urowsp, mrowsp = stage1(idx2, ut, mt, utail, mtail)
    stage2 = _build_stage2(batch)
    out = stage2(urowsp, mrowsp)
    return out.reshape(batch, 1)


# speedup vs baseline: 10.9604x; 1.2801x over previous
import functools

import jax
import jax.numpy as jnp
from jax import lax
from jax.experimental import pallas as pl
from jax.experimental.pallas import tpu as pltpu
from jax.experimental.pallas import tpu_sc as plsc

_EMBED = 32
_CHUNK = 512
_RANGE = 65536
_ROWPAD = 128
_HITBUF = 128


def _popcount(mask):
    return plsc.all_reduce_population_count(mask)[0]


def _build_stage1(batch, n_rows):
    info = plsc.get_sparse_core_info()
    nc, ns, lanes = info.num_cores, info.num_subcores, info.num_lanes
    rows_per_w = _RANGE
    n_full = (n_rows // _CHUNK) * _CHUNK
    tail = n_rows - n_full
    mesh = plsc.VectorSubcoreMesh(core_axis_name="c", subcore_axis_name="s")

    @functools.partial(
        pl.kernel,
        mesh=mesh,
        out_type=(
            jax.ShapeDtypeStruct((batch, _ROWPAD), jnp.float32),
            jax.ShapeDtypeStruct((batch, _ROWPAD), jnp.float32),
        ),
        scratch_types=[
            pltpu.VMEM((2048,), jnp.int32),
            pltpu.VMEM((batch + 16,), jnp.int32),
            pltpu.VMEM((batch + 16,), jnp.int32),
            pltpu.VMEM((2, _EMBED, _CHUNK), jnp.float32),
            pltpu.VMEM((tail, _EMBED), jnp.float32),
            pltpu.VMEM((_HITBUF, _ROWPAD), jnp.float32),
            pltpu.VMEM((_HITBUF + 16,), jnp.int32),
            pltpu.VMEM((_HITBUF,), jnp.int32),
            pltpu.VMEM((batch + 16,), jnp.int32),
            pltpu.VMEM((batch + 16,), jnp.int32),
            pltpu.SemaphoreType.DMA,
        ],
        compiler_params=pltpu.CompilerParams(needs_layout_passes=False),
    )
    def stage1(idx2_hbm, ut_hbm, mt_hbm, utail_hbm, mtail_hbm,
               urowsp_hbm, mrowsp_hbm,
               idx_v, h1i_v, h1s_v, chunk_v, tail_v, hrow_v, hslot_v,
               hslot2_v, h2i_v, h2s_v, sem):
        core = lax.axis_index("c")
        sid = lax.axis_index("s")
        lo = sid * rows_per_w
        hi = jnp.minimum(lo + rows_per_w, n_rows)
        iota = lax.iota(jnp.int32, lanes)

        def run(tab_hbm, tail_hbm, out_hbm):
            pltpu.sync_copy(tail_hbm, tail_v)

            def l1_blk(b, n1_in):
                pltpu.sync_copy(idx2_hbm.at[core, pl.ds(b * 2048, 2048)],
                                idx_v)

                def l1_body(k, n1):
                    v = idx_v[pl.ds(k * lanes, lanes)]
                    slots = b * 2048 + k * lanes + iota
                    mask = (v >= lo) & (v < hi)
                    cnt = _popcount(mask)
                    plsc.store_compressed(
                        h1i_v.at[pl.ds(n1, lanes)], v, mask=mask)
                    plsc.store_compressed(
                        h1s_v.at[pl.ds(n1, lanes)], slots, mask=mask)
                    return n1 + cnt

                return lax.fori_loop(0, 2048 // lanes, l1_body, n1_in)

            n1 = lax.fori_loop(0, batch // 2048, l1_blk, 0)

            def flush(nh):
                @pl.when(nh > 0)
                def _():
                    def pad_body(q, _):
                        cur = hslot_v[pl.ds(q * lanes, lanes)]
                        hslot2_v[pl.ds(q * lanes, lanes)] = jnp.where(
                            q * lanes + iota >= nh, -1, cur)
                        return _

                    lax.fori_loop(0, _HITBUF // lanes, pad_body, 0)
                    pltpu.async_copy(
                        hrow_v,
                        out_hbm.at[plsc.Indices(hslot2_v, ignored_value=-1)],
                        sem).wait()

            n_chunks = rows_per_w // _CHUNK

            def fetch(cc, buf):
                base = lo + cc * _CHUNK

                @pl.when(base < n_full)
                def _():
                    pltpu.async_copy(
                        tab_hbm.at[:, pl.ds(base, _CHUNK)],
                        chunk_v.at[buf], sem)

            def wait_fetch(cc, buf):
                base = lo + cc * _CHUNK

                @pl.when(base < n_full)
                def _():
                    pltpu.make_async_copy(
                        tab_hbm.at[:, pl.ds(lo, _CHUNK)],
                        chunk_v.at[buf], sem).wait()

            def tail_scan(j, nh_in):
                v = h1i_v[pl.ds(j * lanes, lanes)]
                sl = h1s_v[pl.ds(j * lanes, lanes)]
                valid = (j * lanes + iota) < n1
                intail = valid & (v >= n_full)
                i_tl = intail.astype(jnp.int32)

                def extract(k, nh_k):
                    m_tl = i_tl[k] == 1

                    @pl.when(m_tl)
                    def _():
                        rt = v[k] - n_full
                        t0 = plsc.load_gather(
                            tail_v, [jnp.full((lanes,), rt, jnp.int32),
                                     iota])
                        t1 = plsc.load_gather(
                            tail_v, [jnp.full((lanes,), rt, jnp.int32),
                                     iota + lanes])
                        hrow_v[nh_k, pl.ds(0, lanes)] = t0
                        hrow_v[nh_k, pl.ds(lanes, lanes)] = t1
                        plsc.store_compressed(
                            hslot_v.at[pl.ds(nh_k, lanes)], sl,
                            mask=iota == k)

                    return nh_k + jnp.where(m_tl, 1, 0)

                def run_group():
                    nh_out = nh_in
                    for k in range(lanes):
                        nh_out = extract(k, nh_out)

                    @pl.when(nh_out > _HITBUF - lanes)
                    def _():
                        flush(nh_out)

                    return jnp.where(nh_out > _HITBUF - lanes, 0, nh_out)

                return lax.cond(_popcount(intail) > 0, run_group,
                                lambda: nh_in)

            n_j0 = (n1 + lanes - 1) // lanes
            nh_tail = lax.fori_loop(0, n_j0, tail_scan, 0)

            fetch(0, 0)

            def chunk_body(cc, nh):
                base = lo + cc * _CHUNK
                buf = lax.rem(cc, 2)

                @pl.when(cc + 1 < n_chunks)
                def _():
                    fetch(cc + 1, 1 - buf)

                wait_fetch(cc, buf)
                cbuf = chunk_v.at[buf]
                in_range = base < n_full

                def collect(j, n2):
                    v = h1i_v[pl.ds(j * lanes, lanes)]
                    sl = h1s_v[pl.ds(j * lanes, lanes)]
                    valid = (j * lanes + iota) < n1
                    inch = valid & (v >= base) & (v < base + _CHUNK) & in_range
                    cnt = _popcount(inch)
                    plsc.store_compressed(
                        h2i_v.at[pl.ds(n2, lanes)], v - base, mask=inch)
                    plsc.store_compressed(
                        h2s_v.at[pl.ds(n2, lanes)], sl, mask=inch)
                    return n2 + cnt

                n_j = (n1 + lanes - 1) // lanes
                n2 = lax.fori_loop(0, n_j, collect, 0)

                def ext_group(g, nh_in):
                    v = h2i_v[pl.ds(g * lanes, lanes)]
                    sl = h2s_v[pl.ds(g * lanes, lanes)]
                    valid_i = ((g * lanes + iota) < n2).astype(jnp.int32)

                    def extract(k, nh_k):
                        m_ch = valid_i[k] == 1

                        @pl.when(m_ch)
                        def _():
                            rc = v[k]
                            rc_vec = jnp.full((lanes,), rc, jnp.int32)
                            g0 = plsc.load_gather(cbuf, [iota, rc_vec])
                            g1 = plsc.load_gather(
                                cbuf, [iota + lanes, rc_vec])
                            hrow_v[nh_k, pl.ds(0, lanes)] = g0
                            hrow_v[nh_k, pl.ds(lanes, lanes)] = g1
                            plsc.store_compressed(
                                hslot_v.at[pl.ds(nh_k, lanes)], sl,
                                mask=iota == k)

                        return nh_k + jnp.where(m_ch, 1, 0)

                    nh_out = nh_in
                    for k in range(lanes):
                        nh_out = extract(k, nh_out)

                    @pl.when(nh_out > _HITBUF - lanes)
                    def _():
                        flush(nh_out)

                    return jnp.where(nh_out > _HITBUF - lanes, 0, nh_out)

                n_g = (n2 + lanes - 1) // lanes
                return lax.fori_loop(0, n_g, ext_group, nh)

            nh_final = lax.fori_loop(0, n_chunks, chunk_body, nh_tail)
            flush(nh_final)

        @pl.when(core == 0)
        def _():
            run(ut_hbm, utail_hbm, urowsp_hbm)

        @pl.when(core == 1)
        def _():
            run(mt_hbm, mtail_hbm, mrowsp_hbm)

    return stage1


def _build_stage2(batch):
    info = plsc.get_sparse_core_info()
    nc, ns, lanes = info.num_cores, info.num_subcores, info.num_lanes
    nw = nc * ns
    b_per_w = batch // nw
    blk = 128
    mesh = plsc.VectorSubcoreMesh(core_axis_name="c", subcore_axis_name="s")

    @functools.partial(
        pl.kernel,
        mesh=mesh,
        out_type=jax.ShapeDtypeStruct((batch,), jnp.float32),
        scratch_types=[
            pltpu.VMEM((blk, _ROWPAD), jnp.float32),
            pltpu.VMEM((blk, _ROWPAD), jnp.float32),
            pltpu.VMEM((b_per_w,), jnp.float32),
            pltpu.SemaphoreType.DMA,
        ],
        compiler_params=pltpu.CompilerParams(needs_layout_passes=False),
    )
    def stage2(urowsp_hbm, mrowsp_hbm, out_hbm, u_v, m_v, out_v, sem):
        wid = lax.axis_index("s") * nc + lax.axis_index("c")
        base = wid * b_per_w
        iota = lax.iota(jnp.int32, lanes)

        for sub in range(b_per_w // blk):
            pltpu.sync_copy(urowsp_hbm.at[pl.ds(base + sub * blk, blk)], u_v)
            pltpu.sync_copy(mrowsp_hbm.at[pl.ds(base + sub * blk, blk)], m_v)

            def grp_body(g, carry):
                rows = g * lanes + iota
                acc = jnp.zeros((lanes,), jnp.float32)
                for d in range(_EMBED):
                    col = jnp.full((lanes,), d, jnp.int32)
                    acc = acc + (plsc.load_gather(u_v, [rows, col])
                                 * plsc.load_gather(m_v, [rows, col]))
                out_v[pl.ds(sub * blk + g * lanes, lanes)] = acc
                return carry

            lax.fori_loop(0, blk // lanes, grp_body, 0)

        pltpu.sync_copy(out_v, out_hbm.at[pl.ds(base, b_per_w)])

    return stage2


def kernel(user_movie_pair, user_embeddings, movie_embeddings):
    batch = user_movie_pair.shape[0]
    n_rows = user_embeddings.shape[0]
    n_full = (n_rows // _CHUNK) * _CHUNK
    pair = user_movie_pair.astype(jnp.int32)
    idx2 = jnp.stack([pair[:, 0], pair[:, 1]])
    ut = user_embeddings.T
    mt = movie_embeddings.T
    utail = user_embeddings[n_full:]
    mtail = movie_embeddings[n_full:]
    stage1 = _build_stage1(batch, n_rows)
    urowsp, mrowsp = stage1(idx2, ut, mt, utail, mtail)
    stage2 = _build_stage2(batch)
    out = stage2(urowsp, mrowsp)
    return out.reshape(batch, 1)


# speedup vs baseline: 11.6570x; 1.0636x over previous
import functools

import jax
import jax.numpy as jnp
from jax import lax
from jax.experimental import pallas as pl
from jax.experimental.pallas import tpu as pltpu
from jax.experimental.pallas import tpu_sc as plsc

_EMBED = 32
_CHUNK = 512
_RANGE = 65536
_ROWPAD = 128
_HITBUF = 128


def _popcount(mask):
    return plsc.all_reduce_population_count(mask)[0]


def _build_stage1(batch, n_rows):
    info = plsc.get_sparse_core_info()
    nc, ns, lanes = info.num_cores, info.num_subcores, info.num_lanes
    rows_per_w = _RANGE
    n_full = (n_rows // _CHUNK) * _CHUNK
    tail = n_rows - n_full
    mesh = plsc.VectorSubcoreMesh(core_axis_name="c", subcore_axis_name="s")

    @functools.partial(
        pl.kernel,
        mesh=mesh,
        out_type=(
            jax.ShapeDtypeStruct((batch, _ROWPAD), jnp.float32),
            jax.ShapeDtypeStruct((batch, _ROWPAD), jnp.float32),
        ),
        scratch_types=[
            pltpu.VMEM((2048,), jnp.int32),
            pltpu.VMEM((batch + 16,), jnp.int32),
            pltpu.VMEM((batch + 16,), jnp.int32),
            pltpu.VMEM((2, _EMBED, _CHUNK), jnp.float32),
            pltpu.VMEM((tail, _EMBED), jnp.float32),
            pltpu.VMEM((_HITBUF, _ROWPAD), jnp.float32),
            pltpu.VMEM((_HITBUF + 16,), jnp.int32),
            pltpu.VMEM((_HITBUF,), jnp.int32),
            pltpu.VMEM((batch + 16,), jnp.int32),
            pltpu.VMEM((batch + 16,), jnp.int32),
            pltpu.SemaphoreType.DMA,
        ],
        compiler_params=pltpu.CompilerParams(needs_layout_passes=False),
    )
    def stage1(idx2_hbm, ut_hbm, mt_hbm, utail_hbm, mtail_hbm,
               urowsp_hbm, mrowsp_hbm,
               idx_v, h1i_v, h1s_v, chunk_v, tail_v, hrow_v, hslot_v,
               hslot2_v, h2i_v, h2s_v, sem):
        core = lax.axis_index("c")
        sid = lax.axis_index("s")
        lo = sid * rows_per_w
        hi = jnp.minimum(lo + rows_per_w, n_rows)
        iota = lax.iota(jnp.int32, lanes)

        def run(tab_hbm, tail_hbm, out_hbm):
            pltpu.sync_copy(tail_hbm, tail_v)

            def l1_blk(b, n1_in):
                pltpu.sync_copy(idx2_hbm.at[core, pl.ds(b * 2048, 2048)],
                                idx_v)

                def l1_body(k, n1):
                    v = idx_v[pl.ds(k * lanes, lanes)]
                    slots = b * 2048 + k * lanes + iota
                    mask = (v >= lo) & (v < hi)
                    cnt = _popcount(mask)
                    plsc.store_compressed(
                        h1i_v.at[pl.ds(n1, lanes)], v, mask=mask)
                    plsc.store_compressed(
                        h1s_v.at[pl.ds(n1, lanes)], slots, mask=mask)
                    return n1 + cnt

                return lax.fori_loop(0, 2048 // lanes, l1_body, n1_in)

            n1 = lax.fori_loop(0, batch // 2048, l1_blk, 0)

            def flush(nh):
                @pl.when(nh > 0)
                def _():
                    def pad_body(q, _):
                        cur = hslot_v[pl.ds(q * lanes, lanes)]
                        hslot2_v[pl.ds(q * lanes, lanes)] = jnp.where(
                            q * lanes + iota >= nh, -1, cur)
                        return _

                    lax.fori_loop(0, _HITBUF // lanes, pad_body, 0)
                    pltpu.async_copy(
                        hrow_v,
                        out_hbm.at[plsc.Indices(hslot2_v, ignored_value=-1)],
                        sem).wait()

            n_chunks = rows_per_w // _CHUNK

            def fetch(cc, buf):
                base = lo + cc * _CHUNK

                @pl.when(base < n_full)
                def _():
                    pltpu.async_copy(
                        tab_hbm.at[:, pl.ds(base, _CHUNK)],
                        chunk_v.at[buf], sem)

            def wait_fetch(cc, buf):
                base = lo + cc * _CHUNK

                @pl.when(base < n_full)
                def _():
                    pltpu.make_async_copy(
                        tab_hbm.at[:, pl.ds(lo, _CHUNK)],
                        chunk_v.at[buf], sem).wait()

            def tail_scan(j, nh_in):
                v = h1i_v[pl.ds(j * lanes, lanes)]
                sl = h1s_v[pl.ds(j * lanes, lanes)]
                valid = (j * lanes + iota) < n1
                intail = valid & (v >= n_full)
                i_tl = intail.astype(jnp.int32)

                def extract(k, nh_k):
                    m_tl = i_tl[k] == 1

                    @pl.when(m_tl)
                    def _():
                        rt = v[k] - n_full
                        t0 = plsc.load_gather(
                            tail_v, [jnp.full((lanes,), rt, jnp.int32),
                                     iota])
                        t1 = plsc.load_gather(
                            tail_v, [jnp.full((lanes,), rt, jnp.int32),
                                     iota + lanes])
                        hrow_v[nh_k, pl.ds(0, lanes)] = t0
                        hrow_v[nh_k, pl.ds(lanes, lanes)] = t1
                        plsc.store_compressed(
                            hslot_v.at[pl.ds(nh_k, lanes)], sl,
                            mask=iota == k)

                    return nh_k + jnp.where(m_tl, 1, 0)

                def run_group():
                    nh_out = nh_in
                    for k in range(lanes):
                        nh_out = extract(k, nh_out)

                    @pl.when(nh_out > _HITBUF - lanes)
                    def _():
                        flush(nh_out)

                    return jnp.where(nh_out > _HITBUF - lanes, 0, nh_out)

                return lax.cond(_popcount(intail) > 0, run_group,
                                lambda: nh_in)

            n_j0 = (n1 + lanes - 1) // lanes
            nh_tail = lax.fori_loop(0, n_j0, tail_scan, 0)

            fetch(0, 0)

            def chunk_body(cc, nh):
                base = lo + cc * _CHUNK
                buf = lax.rem(cc, 2)

                @pl.when(cc + 1 < n_chunks)
                def _():
                    fetch(cc + 1, 1 - buf)

                wait_fetch(cc, buf)
                cbuf = chunk_v.at[buf]
                in_range = base < n_full

                def collect(j, n2):
                    v = h1i_v[pl.ds(j * lanes, lanes)]
                    sl = h1s_v[pl.ds(j * lanes, lanes)]
                    valid = (j * lanes + iota) < n1
                    inch = valid & (v >= base) & (v < base + _CHUNK) & in_range
                    cnt = _popcount(inch)
                    plsc.store_compressed(
                        h2i_v.at[pl.ds(n2, lanes)], v - base, mask=inch)
                    plsc.store_compressed(
                        h2s_v.at[pl.ds(n2, lanes)], sl, mask=inch)
                    return n2 + cnt

                n_j = (n1 + lanes - 1) // lanes
                n2 = lax.fori_loop(0, n_j, collect, 0)

                def ext_group(g, nh_in):
                    v = h2i_v[pl.ds(g * lanes, lanes)]
                    sl = h2s_v[pl.ds(g * lanes, lanes)]
                    valid_i = ((g * lanes + iota) < n2).astype(jnp.int32)

                    def extract(k, nh_k):
                        m_ch = valid_i[k] == 1

                        @pl.when(m_ch)
                        def _():
                            rc = v[k]
                            rc_vec = jnp.full((lanes,), rc, jnp.int32)
                            g0 = plsc.load_gather(cbuf, [iota, rc_vec])
                            g1 = plsc.load_gather(
                                cbuf, [iota + lanes, rc_vec])
                            hrow_v[nh_k, pl.ds(0, lanes)] = g0
                            hrow_v[nh_k, pl.ds(lanes, lanes)] = g1
                            plsc.store_compressed(
                                hslot_v.at[pl.ds(nh_k, lanes)], sl,
                                mask=iota == k)

                        return nh_k + jnp.where(m_ch, 1, 0)

                    nh_out = nh_in
                    for k in range(lanes):
                        nh_out = extract(k, nh_out)

                    @pl.when(nh_out > _HITBUF - lanes)
                    def _():
                        flush(nh_out)

                    return jnp.where(nh_out > _HITBUF - lanes, 0, nh_out)

                n_g = (n2 + lanes - 1) // lanes
                return lax.fori_loop(0, n_g, ext_group, nh)

            nh_final = lax.fori_loop(0, n_chunks, chunk_body, nh_tail)
            flush(nh_final)

        @pl.when(core == 0)
        def _():
            run(ut_hbm, utail_hbm, urowsp_hbm)

        @pl.when(core == 1)
        def _():
            run(mt_hbm, mtail_hbm, mrowsp_hbm)

    return stage1


def _build_stage2(batch):
    info = plsc.get_sparse_core_info()
    nc, ns, lanes = info.num_cores, info.num_subcores, info.num_lanes
    nw = nc * ns
    b_per_w = batch // nw
    blk = 128
    mesh = plsc.VectorSubcoreMesh(core_axis_name="c", subcore_axis_name="s")

    @functools.partial(
        pl.kernel,
        mesh=mesh,
        out_type=jax.ShapeDtypeStruct((batch,), jnp.float32),
        scratch_types=[
            pltpu.VMEM((blk, _ROWPAD), jnp.float32),
            pltpu.VMEM((blk, _ROWPAD), jnp.float32),
            pltpu.VMEM((b_per_w,), jnp.float32),
            pltpu.SemaphoreType.DMA,
        ],
        compiler_params=pltpu.CompilerParams(needs_layout_passes=False),
    )
    def stage2(urowsp_hbm, mrowsp_hbm, out_hbm, u_v, m_v, out_v, sem):
        wid = lax.axis_index("s") * nc + lax.axis_index("c")
        base = wid * b_per_w
        iota = lax.iota(jnp.int32, lanes)

        for sub in range(b_per_w // blk):
            pltpu.sync_copy(urowsp_hbm.at[pl.ds(base + sub * blk, blk)], u_v)
            pltpu.sync_copy(mrowsp_hbm.at[pl.ds(base + sub * blk, blk)], m_v)

            def grp_body(g, carry):
                rows = g * lanes + iota
                acc = jnp.zeros((lanes,), jnp.float32)
                for d in range(_EMBED):
                    col = (iota + d) & (_EMBED - 1)
                    acc = acc + (plsc.load_gather(u_v, [rows, col])
                                 * plsc.load_gather(m_v, [rows, col]))
                out_v[pl.ds(sub * blk + g * lanes, lanes)] = acc
                return carry

            lax.fori_loop(0, blk // lanes, grp_body, 0)

        pltpu.sync_copy(out_v, out_hbm.at[pl.ds(base, b_per_w)])

    return stage2


def kernel(user_movie_pair, user_embeddings, movie_embeddings):
    batch = user_movie_pair.shape[0]
    n_rows = user_embeddings.shape[0]
    n_full = (n_rows // _CHUNK) * _CHUNK
    pair = user_movie_pair.astype(jnp.int32)
    idx2 = jnp.stack([pair[:, 0], pair[:, 1]])
    ut = user_embeddings.T
    mt = movie_embeddings.T
    utail = user_embeddings[n_full:]
    mtail = movie_embeddings[n_full:]
    stage1 = _build_stage1(batch, n_rows)
    urowsp, mrowsp = stage1(idx2, ut, mt, utail, mtail)
    stage2 = _build_stage2(batch)
    out = stage2(urowsp, mrowsp)
    return out.reshape(batch, 1)
